# Initial kernel scaffold; baseline (speedup 1.0000x reference)
import jax
import jax.numpy as jnp
from jax.experimental import pallas as pl


def kernel(adj_indices, adj_values, user_emb, item_emb):
    raise NotImplementedError("write your pallas kernel here")


# speedup vs baseline: 2.5496x; 2.5496x over previous
import functools

import jax
import jax.numpy as jnp
from jax import lax
from jax.experimental import pallas as pl
from jax.experimental.pallas import tpu as pltpu
from jax.experimental.pallas import tpu_sc as plsc

N_USERS = 25000
N_ITEMS = 25000
D = 64
N_LAYERS = 3
E = 800000

NC = 2
NS = 16
L = 16

CLEN = 128
SUB = 8
K = SUB * CLEN
CH = 49
E_PAD = NS * CH * K

R_TILE = 1600
HALF = NS * R_TILE
X_ROWS = NC * HALF
TRASH = N_USERS


def _layer_body(x_hbm, src_hbm, dl_hbm, vals_hbm, out_hbm,
                src_v, dli_v, vals_v, rows_v, acc_sh, gsem):
    s = lax.axis_index("c")
    t = lax.axis_index("s")

    def zrow(i, c):
        for k in range(D // L):
            rows_v[i, pl.ds(k * L, L)] = jnp.zeros((L,), jnp.float32)
        return c
    lax.fori_loop(0, CLEN, zrow, 0)
    base_acc = t * R_TILE
    for i in range(R_TILE // CLEN):
        pltpu.sync_copy(rows_v, acc_sh.at[pl.ds(base_acc + i * CLEN, CLEN)])
    rem = R_TILE % CLEN
    if rem:
        pltpu.sync_copy(rows_v.at[pl.ds(0, rem)],
                        acc_sh.at[pl.ds(base_acc + (R_TILE // CLEN) * CLEN, rem)])
    plsc.subcore_barrier()

    def chunk(c, carry):
        ci = t * CH + c
        pltpu.sync_copy(src_hbm.at[ci], src_v)
        pltpu.sync_copy(dl_hbm.at[s, ci], dli_v)
        pltpu.sync_copy(vals_hbm.at[ci], vals_v)
        for j in range(SUB):
            pltpu.async_copy(x_hbm.at[src_v.at[j]], rows_v, gsem).wait()

            def scale16(g, cc):
                v16 = vals_v[pl.ds(j * CLEN + g * L, L)]
                for e in range(L):
                    r = g * L + e
                    v = v16[e]
                    for k in range(D // L):
                        sl = pl.ds(k * L, L)
                        rows_v[r, sl] = rows_v[r, sl] * v
                return cc
            lax.fori_loop(0, CLEN // L, scale16, 0)
            pltpu.sync_copy(rows_v, acc_sh.at[dli_v.at[j]], add=True)
        return carry
    lax.fori_loop(0, CH, chunk, 0)

    plsc.subcore_barrier()
    pltpu.sync_copy(acc_sh.at[pl.ds(base_acc, R_TILE)],
                    out_hbm.at[pl.ds(s * HALF + base_acc, R_TILE)])


_layer = functools.partial(
    pl.kernel,
    out_type=jax.ShapeDtypeStruct((X_ROWS, D), jnp.float32),
    mesh=plsc.VectorSubcoreMesh(core_axis_name="c", subcore_axis_name="s",
                                num_cores=NC, num_subcores=NS),
    scratch_types=[
        pltpu.VMEM((SUB, CLEN), jnp.int32),
        pltpu.VMEM((SUB, CLEN), jnp.int32),
        pltpu.VMEM((K,), jnp.float32),
        pltpu.VMEM((CLEN, D), jnp.float32),
        pltpu.VMEM_SHARED((HALF, D), jnp.float32),
        pltpu.SemaphoreType.DMA,
    ],
    compiler_params=pltpu.CompilerParams(use_tc_tiling_on_sc=False),
)(_layer_body)


def kernel(adj_indices, adj_values, user_emb, item_emb):
    dst = adj_indices[0].astype(jnp.int32)
    src = adj_indices[1].astype(jnp.int32)
    vals = adj_values.astype(jnp.float32)

    pad = E_PAD - E
    dst = jnp.concatenate([dst, jnp.zeros((pad,), jnp.int32)])
    src = jnp.concatenate([src, jnp.zeros((pad,), jnp.int32)])
    vals = jnp.concatenate([vals, jnp.zeros((pad,), jnp.float32)])

    srcp = src + jnp.where(src >= N_USERS, HALF - N_USERS, 0).astype(jnp.int32)
    dl0 = jnp.where(dst < N_USERS, dst, TRASH).astype(jnp.int32)
    dl1 = jnp.where(dst >= N_USERS, dst - N_USERS, TRASH).astype(jnp.int32)

    src3 = srcp.reshape(NS * CH, SUB, CLEN)
    dl4 = jnp.stack([dl0, dl1]).reshape(NC, NS * CH, SUB, CLEN)
    vals2 = vals.reshape(NS * CH, K)

    zpad = jnp.zeros((HALF - N_USERS, D), jnp.float32)
    x = jnp.concatenate([user_emb, zpad, item_emb, zpad], axis=0)

    acc = x
    for _ in range(N_LAYERS):
        x = _layer(x, src3, dl4, vals2)
        acc = acc + x
    out = acc * (1.0 / (N_LAYERS + 1))
    return (out[:N_USERS], out[HALF:HALF + N_ITEMS])
